# baseline (speedup 1.0000x reference)
import functools

import jax
import jax.numpy as jnp
from jax import lax
from jax.experimental import pallas as pl
from jax.experimental.pallas import tpu as pltpu
from jax.experimental.pallas import tpu_sc as plsc

_NC = 2
_NS = 16
_NW = _NC * _NS
_R = 16
_SC_ROWS = 4096
_BB = 128


def _sc_body(row, bpw, cls_hbm, emb_hbm, out_hbm, buf, sem):
    wid = lax.axis_index("s") * _NC + lax.axis_index("c")
    d = 64
    for r in range(_R):
        pltpu.sync_copy(cls_hbm, buf.at[pl.ds(r * row, d)])
        pltpu.sync_copy(emb_hbm, buf.at[pl.ds(r * row + d, row - d)])
    base = wid * bpw * row
    chunk = _R * row
    copies = [
        pltpu.make_async_copy(buf, out_hbm.at[pl.ds(base + c * chunk, chunk)], sem)
        for c in range(bpw // _R)
    ]
    for cp in copies:
        cp.start()
    for cp in copies:
        cp.wait()


def _tc_body(comb_ref, alias_ref, out_ref):
    del alias_ref
    out_ref[...] = jnp.broadcast_to(comb_ref[...], out_ref.shape)


def kernel(x, feature_embeddings, cls_token):
    batch = x.shape[0]
    num_feats, d = feature_embeddings.shape
    seq = num_feats + 1
    row = seq * d
    bpw = _SC_ROWS // _NW

    mesh = plsc.VectorSubcoreMesh(core_axis_name="c", subcore_axis_name="s")
    sc_fill = pl.kernel(
        functools.partial(_sc_body, row, bpw),
        out_type=jax.ShapeDtypeStruct((batch * row,), jnp.float32),
        mesh=mesh,
        scratch_types=[
            pltpu.VMEM((_R * row,), jnp.float32),
            pltpu.SemaphoreType.DMA,
        ],
    )
    sc_out = sc_fill(cls_token.reshape(d), feature_embeddings.reshape(num_feats * d))
    sc_out = sc_out.reshape(batch, row)

    comb = jnp.concatenate([cls_token[0], feature_embeddings], axis=0)
    comb_flat = comb.reshape(1, row)
    off_blocks = _SC_ROWS // _BB
    out2d = pl.pallas_call(
        _tc_body,
        grid=((batch - _SC_ROWS) // _BB,),
        in_specs=[
            pl.BlockSpec((1, row), lambda i: (0, 0)),
            pl.BlockSpec(memory_space=pl.ANY),
        ],
        out_specs=pl.BlockSpec((_BB, row), lambda i: (i + off_blocks, 0)),
        out_shape=jax.ShapeDtypeStruct((batch, row), jnp.float32),
        input_output_aliases={1: 0},
    )(comb_flat, sc_out)
    return out2d.reshape(batch, seq, d)


# speedup vs baseline: 1.8243x; 1.8243x over previous
import functools

import jax
import jax.numpy as jnp
from jax import lax
from jax.experimental import pallas as pl
from jax.experimental.pallas import tpu as pltpu
from jax.experimental.pallas import tpu_sc as plsc

_NC = 2
_NS = 16
_NW = _NC * _NS
_R = 16
_SC_ROWS = 4096
_BB = 128


def _sc_body(bpw, combr_hbm, out_hbm, buf, sem):
    wid = lax.axis_index("s") * _NC + lax.axis_index("c")
    pltpu.sync_copy(combr_hbm, buf)
    base = wid * bpw
    copies = [
        pltpu.make_async_copy(buf, out_hbm.at[pl.ds(base + c * _R, _R), :], sem)
        for c in range(bpw // _R)
    ]
    for cp in copies:
        cp.start()
    for cp in copies:
        cp.wait()


def _tc_body(comb_ref, alias_ref, out_ref):
    del alias_ref
    out_ref[...] = jnp.broadcast_to(comb_ref[...], out_ref.shape)


def kernel(x, feature_embeddings, cls_token):
    batch = x.shape[0]
    num_feats, d = feature_embeddings.shape
    seq = num_feats + 1
    row = seq * d
    bpw = _SC_ROWS // _NW

    comb_flat = jnp.concatenate([cls_token[0], feature_embeddings], axis=0).reshape(1, row)
    comb_r = jnp.broadcast_to(comb_flat, (_R, row))

    mesh = plsc.VectorSubcoreMesh(core_axis_name="c", subcore_axis_name="s")
    sc_fill = pl.kernel(
        functools.partial(_sc_body, bpw),
        out_type=jax.ShapeDtypeStruct((batch, row), jnp.float32),
        mesh=mesh,
        scratch_types=[
            pltpu.VMEM((_R, row), jnp.float32),
            pltpu.SemaphoreType.DMA,
        ],
    )
    sc_out = sc_fill(comb_r)

    off_blocks = _SC_ROWS // _BB
    out2d = pl.pallas_call(
        _tc_body,
        grid=((batch - _SC_ROWS) // _BB,),
        in_specs=[
            pl.BlockSpec((1, row), lambda i: (0, 0)),
            pl.BlockSpec(memory_space=pl.ANY),
        ],
        out_specs=pl.BlockSpec((_BB, row), lambda i: (i + off_blocks, 0)),
        out_shape=jax.ShapeDtypeStruct((batch, row), jnp.float32),
        input_output_aliases={1: 0},
    )(comb_flat, sc_out)
    return out2d.reshape(batch, seq, d)


# speedup vs baseline: 1.8290x; 1.0026x over previous
import functools

import jax
import jax.numpy as jnp
from jax import lax
from jax.experimental import pallas as pl
from jax.experimental.pallas import tpu as pltpu
from jax.experimental.pallas import tpu_sc as plsc

_NC = 2
_NS = 16
_NW = _NC * _NS
_R = 16
_SC_ROWS = 8192
_BB = 128


def _sc_body(bpw, combr_hbm, out_hbm, buf, sem):
    wid = lax.axis_index("s") * _NC + lax.axis_index("c")
    pltpu.sync_copy(combr_hbm, buf)
    base = wid * bpw
    copies = [
        pltpu.make_async_copy(buf, out_hbm.at[pl.ds(base + c * _R, _R), :], sem)
        for c in range(bpw // _R)
    ]
    for cp in copies:
        cp.start()
    for cp in copies:
        cp.wait()


def _tc_body(comb_ref, alias_ref, out_ref):
    del alias_ref
    out_ref[...] = jnp.broadcast_to(comb_ref[...], out_ref.shape)


def kernel(x, feature_embeddings, cls_token):
    batch = x.shape[0]
    num_feats, d = feature_embeddings.shape
    seq = num_feats + 1
    row = seq * d
    bpw = _SC_ROWS // _NW

    comb_flat = jnp.concatenate([cls_token[0], feature_embeddings], axis=0).reshape(1, row)
    comb_r = jnp.broadcast_to(comb_flat, (_R, row))

    mesh = plsc.VectorSubcoreMesh(core_axis_name="c", subcore_axis_name="s")
    sc_fill = pl.kernel(
        functools.partial(_sc_body, bpw),
        out_type=jax.ShapeDtypeStruct((batch, row), jnp.float32),
        mesh=mesh,
        scratch_types=[
            pltpu.VMEM((_R, row), jnp.float32),
            pltpu.SemaphoreType.DMA,
        ],
    )
    sc_out = sc_fill(comb_r)

    off_blocks = _SC_ROWS // _BB
    out2d = pl.pallas_call(
        _tc_body,
        grid=((batch - _SC_ROWS) // _BB,),
        in_specs=[
            pl.BlockSpec((1, row), lambda i: (0, 0)),
            pl.BlockSpec(memory_space=pl.ANY),
        ],
        out_specs=pl.BlockSpec((_BB, row), lambda i: (i + off_blocks, 0)),
        out_shape=jax.ShapeDtypeStruct((batch, row), jnp.float32),
        input_output_aliases={1: 0},
    )(comb_flat, sc_out)
    return out2d.reshape(batch, seq, d)
